# Initial kernel scaffold; baseline (speedup 1.0000x reference)
import jax
import jax.numpy as jnp
from jax.experimental import pallas as pl


def kernel(z, edge_index, edge_type, rel_emb):
    raise NotImplementedError("write your pallas kernel here")


# speedup vs baseline: 1.2998x; 1.2998x over previous
import functools

import jax
import jax.numpy as jnp
from jax import lax
from jax.experimental import pallas as pl
from jax.experimental.pallas import tpu as pltpu
from jax.experimental.pallas import tpu_sc as plsc

NUM_NODES = 10000
NUM_EDGES = 320000
NUM_RELATIONS = 1000
HIDDEN = 128

NC = 2
NS = 16
L = 16
NW = NC * NS
EPW = NUM_EDGES // NW
B = 80
NCH = EPW // B
NG = B // L


def _norm_body(z_ref, o_ref):
    x = z_ref[...]
    n = jnp.sum(jnp.abs(x), axis=1, keepdims=True)
    o_ref[...] = x / jnp.maximum(n, 1e-12)


def _l1_normalize_rows(z):
    return pl.pallas_call(
        _norm_body,
        out_shape=jax.ShapeDtypeStruct((NUM_NODES, HIDDEN), jnp.float32),
        grid=(5,),
        in_specs=[pl.BlockSpec((NUM_NODES // 5, HIDDEN), lambda i: (i, 0))],
        out_specs=pl.BlockSpec((NUM_NODES // 5, HIDDEN), lambda i: (i, 0)),
    )(z)


def _sc_body(znorm_hbm, rel_hbm, hidx_hbm, tidx_hbm, ridx_hbm, out_hbm,
             hidx_v, tidx_v, ridx_v, hrows_v, trows_v, rrows_v, out_v,
             sem1, sem2, sem3):
    wid = lax.axis_index("s") * NC + lax.axis_index("c")
    pltpu.sync_copy(hidx_hbm.at[pl.ds(wid * EPW, EPW)], hidx_v)
    pltpu.sync_copy(tidx_hbm.at[pl.ds(wid * EPW, EPW)], tidx_v)
    pltpu.sync_copy(ridx_hbm.at[pl.ds(wid * EPW, EPW)], ridx_v)

    row16 = lax.iota(jnp.int32, L)

    def chunk(i, _):
        cp1 = pltpu.async_copy(znorm_hbm.at[hidx_v.at[pl.ds(i * B, B)]],
                               hrows_v, sem1)
        cp2 = pltpu.async_copy(znorm_hbm.at[tidx_v.at[pl.ds(i * B, B)]],
                               trows_v, sem2)
        cp3 = pltpu.async_copy(rel_hbm.at[ridx_v.at[pl.ds(i * B, B)]],
                               rrows_v, sem3)
        cp1.wait()
        cp2.wait()
        cp3.wait()
        for g in range(NG):
            rows = row16 + (g * L)

            def dstep(d, carry):
                acc, col = carry
                h = plsc.load_gather(hrows_v, [rows, col])
                t = plsc.load_gather(trows_v, [rows, col])
                r = plsc.load_gather(rrows_v, [rows, col])
                acc = acc + jnp.abs(h + r - t)
                return acc, col + 1

            acc0 = jnp.zeros((L,), jnp.float32)
            col0 = jnp.zeros((L,), jnp.int32)
            acc, _ = lax.fori_loop(0, HIDDEN, dstep, (acc0, col0))
            out_v[pl.ds(i * B + g * L, L)] = -acc
        return 0

    lax.fori_loop(0, NCH, chunk, 0)
    pltpu.sync_copy(out_v, out_hbm.at[pl.ds(wid * EPW, EPW)])


@functools.partial(jax.jit, static_argnames=())
def _sc_score(znorm, rel_emb, hidx, tidx, ridx):
    mesh = plsc.VectorSubcoreMesh(core_axis_name="c", subcore_axis_name="s",
                                  num_cores=NC, num_subcores=NS)
    return pl.kernel(
        _sc_body,
        out_type=jax.ShapeDtypeStruct((NUM_EDGES,), jnp.float32),
        mesh=mesh,
        compiler_params=pltpu.CompilerParams(needs_layout_passes=False),
        scratch_types=[
            pltpu.VMEM((EPW,), jnp.int32),
            pltpu.VMEM((EPW,), jnp.int32),
            pltpu.VMEM((EPW,), jnp.int32),
            pltpu.VMEM((B, HIDDEN), jnp.float32),
            pltpu.VMEM((B, HIDDEN), jnp.float32),
            pltpu.VMEM((B, HIDDEN), jnp.float32),
            pltpu.VMEM((EPW,), jnp.float32),
            pltpu.SemaphoreType.DMA,
            pltpu.SemaphoreType.DMA,
            pltpu.SemaphoreType.DMA,
        ],
    )(znorm, rel_emb, hidx, tidx, ridx)


def kernel(z, edge_index, edge_type, rel_emb):
    znorm = _l1_normalize_rows(z)
    hidx = edge_index[0].astype(jnp.int32)
    tidx = edge_index[1].astype(jnp.int32)
    ridx = edge_type.astype(jnp.int32)
    return _sc_score(znorm, rel_emb, hidx, tidx, ridx)


# speedup vs baseline: 1.6215x; 1.2475x over previous
import functools

import jax
import jax.numpy as jnp
from jax import lax
from jax.experimental import pallas as pl
from jax.experimental.pallas import tpu as pltpu
from jax.experimental.pallas import tpu_sc as plsc

NUM_NODES = 10000
NUM_EDGES = 320000
NUM_RELATIONS = 1000
HIDDEN = 128

NC = 2
NS = 16
L = 16
NW = NC * NS
EPW = NUM_EDGES // NW
B = 80
NCH = EPW // B
NG = B // L
U = 32


def _norm_body(z_ref, o_ref):
    x = z_ref[...]
    n = jnp.sum(jnp.abs(x), axis=1, keepdims=True)
    o_ref[...] = x / jnp.maximum(n, 1e-12)


def _l1_normalize_rows(z):
    return pl.pallas_call(
        _norm_body,
        out_shape=jax.ShapeDtypeStruct((NUM_NODES, HIDDEN), jnp.float32),
        grid=(5,),
        in_specs=[pl.BlockSpec((NUM_NODES // 5, HIDDEN), lambda i: (i, 0))],
        out_specs=pl.BlockSpec((NUM_NODES // 5, HIDDEN), lambda i: (i, 0)),
    )(z)


def _sc_body(znorm_hbm, rel_hbm, hidx_hbm, tidx_hbm, ridx_hbm, out_hbm,
             hidx_v, tidx_v, ridx_v,
             h0, t0, r0, h1, t1, r1, out_v, s0, s1):
    wid = lax.axis_index("s") * NC + lax.axis_index("c")
    pltpu.sync_copy(hidx_hbm.at[pl.ds(wid * EPW, EPW)], hidx_v)
    pltpu.sync_copy(tidx_hbm.at[pl.ds(wid * EPW, EPW)], tidx_v)
    pltpu.sync_copy(ridx_hbm.at[pl.ds(wid * EPW, EPW)], ridx_v)

    row16 = lax.iota(jnp.int32, L)

    def issue(i, hb, tb, rb, sem):
        pltpu.async_copy(znorm_hbm.at[hidx_v.at[pl.ds(i * B, B)]], hb, sem)
        pltpu.async_copy(znorm_hbm.at[tidx_v.at[pl.ds(i * B, B)]], tb, sem)
        pltpu.async_copy(rel_hbm.at[ridx_v.at[pl.ds(i * B, B)]], rb, sem)

    def drain(hb, tb, rb, sem):
        pltpu.make_async_copy(znorm_hbm.at[pl.ds(0, B)], hb, sem).wait()
        pltpu.make_async_copy(znorm_hbm.at[pl.ds(0, B)], tb, sem).wait()
        pltpu.make_async_copy(rel_hbm.at[pl.ds(0, B)], rb, sem).wait()

    def compute(i, hb, tb, rb):
        for g in range(NG):
            rows = row16 + (g * L)
            z16f = jnp.zeros((L,), jnp.float32)
            z16i = jnp.zeros((L,), jnp.int32)

            def dblk(s, carry):
                a0, a1, col = carry
                for u in range(U):
                    cu = col + u
                    h = plsc.load_gather(hb, [rows, cu])
                    t = plsc.load_gather(tb, [rows, cu])
                    r = plsc.load_gather(rb, [rows, cu])
                    v = jnp.abs(h + r - t)
                    if u % 2 == 0:
                        a0 = a0 + v
                    else:
                        a1 = a1 + v
                return a0, a1, col + U

            a0, a1, _ = lax.fori_loop(0, HIDDEN // U, dblk,
                                      (z16f, z16f, z16i))
            out_v[pl.ds(i * B + g * L, L)] = -(a0 + a1)

    issue(0, h0, t0, r0, s0)

    def pair(k, _):
        i = k * 2
        issue(i + 1, h1, t1, r1, s1)
        drain(h0, t0, r0, s0)
        compute(i, h0, t0, r0)
        issue(i + 2, h0, t0, r0, s0)
        drain(h1, t1, r1, s1)
        compute(i + 1, h1, t1, r1)
        return 0

    lax.fori_loop(0, (NCH - 1) // 2, pair, 0)
    drain(h0, t0, r0, s0)
    compute(NCH - 1, h0, t0, r0)
    pltpu.sync_copy(out_v, out_hbm.at[pl.ds(wid * EPW, EPW)])


@jax.jit
def _sc_score(znorm, rel_emb, hidx, tidx, ridx):
    mesh = plsc.VectorSubcoreMesh(core_axis_name="c", subcore_axis_name="s",
                                  num_cores=NC, num_subcores=NS)
    return pl.kernel(
        _sc_body,
        out_type=jax.ShapeDtypeStruct((NUM_EDGES,), jnp.float32),
        mesh=mesh,
        compiler_params=pltpu.CompilerParams(needs_layout_passes=False),
        scratch_types=[
            pltpu.VMEM((EPW,), jnp.int32),
            pltpu.VMEM((EPW,), jnp.int32),
            pltpu.VMEM((EPW,), jnp.int32),
            pltpu.VMEM((B, HIDDEN), jnp.float32),
            pltpu.VMEM((B, HIDDEN), jnp.float32),
            pltpu.VMEM((B, HIDDEN), jnp.float32),
            pltpu.VMEM((B, HIDDEN), jnp.float32),
            pltpu.VMEM((B, HIDDEN), jnp.float32),
            pltpu.VMEM((B, HIDDEN), jnp.float32),
            pltpu.VMEM((EPW,), jnp.float32),
            pltpu.SemaphoreType.DMA,
            pltpu.SemaphoreType.DMA,
        ],
    )(znorm, rel_emb, hidx, tidx, ridx)


def kernel(z, edge_index, edge_type, rel_emb):
    znorm = _l1_normalize_rows(z)
    hidx = edge_index[0].astype(jnp.int32)
    tidx = edge_index[1].astype(jnp.int32)
    ridx = edge_type.astype(jnp.int32)
    return _sc_score(znorm, rel_emb, hidx, tidx, ridx)
